# speedup vs baseline: 5.1785x; 5.1785x over previous
import functools

import jax
import jax.numpy as jnp
from jax import lax
from jax.experimental import pallas as pl
from jax.experimental.pallas import tpu as pltpu
from jax.experimental.pallas import tpu_sc as plsc

_NC = 2
_NS = 16
_NW = _NC * _NS

_CHUNK = 128


def _gather_rows(idx_flat, table):
    n = idx_flat.shape[0]
    d = table.shape[1]
    per_w = n // _NW
    nchunks = per_w // _CHUNK

    mesh = plsc.VectorSubcoreMesh(core_axis_name="c", subcore_axis_name="s")

    @functools.partial(
        pl.kernel,
        out_type=jax.ShapeDtypeStruct((n, d), jnp.float32),
        mesh=mesh,
        scratch_types=[
            pltpu.VMEM((_CHUNK,), jnp.int32),
            pltpu.VMEM((_CHUNK, d), jnp.float32),
            pltpu.SemaphoreType.DMA,
        ],
    )
    def body(idx_hbm, table_hbm, out_hbm, idx_v, rows_v, sem):
        wid = lax.axis_index("s") * _NC + lax.axis_index("c")
        base = wid * per_w

        def step(g, carry):
            off = base + g * _CHUNK
            pltpu.sync_copy(idx_hbm.at[pl.ds(off, _CHUNK)], idx_v)
            pltpu.async_copy(table_hbm.at[idx_v], rows_v, sem).wait()
            pltpu.sync_copy(rows_v, out_hbm.at[pl.ds(off, _CHUNK)])
            return carry

        lax.fori_loop(0, nchunks, step, 0)

    return body(idx_flat, table)


def kernel(input_ids, embedding_table):
    b, h = input_ids.shape
    idx_flat = input_ids.reshape(-1).astype(jnp.int32)
    out = _gather_rows(idx_flat, embedding_table)
    return out.reshape(b, h, embedding_table.shape[1])


# speedup vs baseline: 9.2050x; 1.7775x over previous
import functools

import jax
import jax.numpy as jnp
from jax import lax
from jax.experimental import pallas as pl
from jax.experimental.pallas import tpu as pltpu
from jax.experimental.pallas import tpu_sc as plsc

_NC = 2
_NS = 16
_NW = _NC * _NS

_CHUNK = 128
_NBUF = 4


def _gather_rows(idx_flat, table):
    n = idx_flat.shape[0]
    d = table.shape[1]
    per_w = n // _NW
    nchunks = per_w // _CHUNK
    n_outer = nchunks // _NBUF

    mesh = plsc.VectorSubcoreMesh(core_axis_name="c", subcore_axis_name="s")

    @functools.partial(
        pl.kernel,
        out_type=jax.ShapeDtypeStruct((n, d), jnp.float32),
        mesh=mesh,
        scratch_types=[
            pltpu.VMEM((per_w,), jnp.int32),
            [pltpu.VMEM((_CHUNK, d), jnp.float32) for _ in range(_NBUF)],
            [pltpu.SemaphoreType.DMA for _ in range(_NBUF)],
        ],
    )
    def body(idx_hbm, table_hbm, out_hbm, idx_all, rows, gsem):
        wid = lax.axis_index("s") * _NC + lax.axis_index("c")
        base = wid * per_w

        pltpu.sync_copy(idx_hbm.at[pl.ds(base, per_w)], idx_all)

        def fire(g, b):
            idx_ref = idx_all.at[pl.ds(g * _CHUNK, _CHUNK)]
            pltpu.async_copy(table_hbm.at[idx_ref], rows[b], gsem[b])

        for b in range(_NBUF):
            fire(b, b)

        def outer(go, carry):
            for b in range(_NBUF):
                g = go * _NBUF + b
                pltpu.make_async_copy(
                    table_hbm.at[pl.ds(0, _CHUNK)], rows[b], gsem[b]
                ).wait()
                pltpu.sync_copy(rows[b], out_hbm.at[pl.ds(base + g * _CHUNK, _CHUNK)])

                @pl.when(go < n_outer - 1)
                def _():
                    fire(g + _NBUF, b)

            return carry

        lax.fori_loop(0, n_outer, outer, 0)

    return body(idx_flat, table)


def kernel(input_ids, embedding_table):
    b, h = input_ids.shape
    idx_flat = input_ids.reshape(-1).astype(jnp.int32)
    out = _gather_rows(idx_flat, embedding_table)
    return out.reshape(b, h, embedding_table.shape[1])


# speedup vs baseline: 9.2529x; 1.0052x over previous
import functools

import jax
import jax.numpy as jnp
from jax import lax
from jax.experimental import pallas as pl
from jax.experimental.pallas import tpu as pltpu
from jax.experimental.pallas import tpu_sc as plsc

_NC = 2
_NS = 16
_NW = _NC * _NS

_CHUNK = 128
_NBUF = 5
_K = 3


def _gather_rows(idx_flat, table):
    n = idx_flat.shape[0]
    d = table.shape[1]
    per_w = n // _NW
    nchunks = per_w // _CHUNK
    n_outer = nchunks // _NBUF

    mesh = plsc.VectorSubcoreMesh(core_axis_name="c", subcore_axis_name="s")

    @functools.partial(
        pl.kernel,
        out_type=jax.ShapeDtypeStruct((n, d), jnp.float32),
        mesh=mesh,
        scratch_types=[
            pltpu.VMEM((per_w,), jnp.int32),
            [pltpu.VMEM((_CHUNK, d), jnp.float32) for _ in range(_NBUF)],
            [pltpu.SemaphoreType.DMA for _ in range(_NBUF)],
            [pltpu.SemaphoreType.DMA for _ in range(_NBUF)],
        ],
    )
    def body(idx_hbm, table_hbm, out_hbm, idx_all, rows, gsem, ssem):
        wid = lax.axis_index("s") * _NC + lax.axis_index("c")
        base = wid * per_w

        pltpu.sync_copy(idx_hbm.at[pl.ds(base, per_w)], idx_all)

        def fire(c, b):
            idx_ref = idx_all.at[pl.ds(c * _CHUNK, _CHUNK)]
            pltpu.async_copy(table_hbm.at[idx_ref], rows[b], gsem[b])

        def wait_gather(b):
            pltpu.make_async_copy(
                table_hbm.at[pl.ds(0, _CHUNK)], rows[b], gsem[b]
            ).wait()

        def wait_scatter(b):
            pltpu.make_async_copy(
                rows[b], out_hbm.at[pl.ds(0, _CHUNK)], ssem[b]
            ).wait()

        for c in range(_K):
            fire(c, c % _NBUF)

        def outer(go, carry):
            for b in range(_NBUF):
                g = go * _NBUF + b
                c = g + _K
                bf = (b + _K) % _NBUF

                @pl.when(g < nchunks - _K)
                def _():
                    @pl.when(g >= _NBUF - _K)
                    def _():
                        wait_scatter(bf)

                    fire(c, bf)

                wait_gather(b)
                pltpu.async_copy(
                    rows[b], out_hbm.at[pl.ds(base + g * _CHUNK, _CHUNK)], ssem[b]
                )

            return carry

        lax.fori_loop(0, n_outer, outer, 0)

        for b in range(_NBUF):
            wait_scatter(b)

    return body(idx_flat, table)


def kernel(input_ids, embedding_table):
    b, h = input_ids.shape
    idx_flat = input_ids.reshape(-1).astype(jnp.int32)
    out = _gather_rows(idx_flat, embedding_table)
    return out.reshape(b, h, embedding_table.shape[1])
